# baseline (speedup 1.0000x reference)
import jax
import jax.numpy as jnp
from jax.experimental import pallas as pl
from jax.experimental.pallas import tpu as pltpu

_TOP_K = 8
_BLOCK_T = 4096


def _gate_kernel(x_ref, w_ref, b_ref, wout_ref, iout_ref, logits_ref):
    j = pl.program_id(0)
    nblk = pl.num_programs(0) - 1

    @pl.when(j < nblk)
    def _matmul():
        logits_ref[jax.lax.rem(j, 2)] = jax.lax.dot_general(
            w_ref[...], x_ref[...], (((1,), (1,)), ((), ())),
            preferred_element_type=jnp.float32,
        )

    @pl.when(j > 0)
    def _select():
        logits = logits_ref[jax.lax.rem(j + 1, 2)]
        sig = jax.nn.sigmoid(logits)
        work = logits + b_ref[...]
        n_exp = work.shape[0]
        iota_f = jax.lax.broadcasted_iota(jnp.int32, work.shape, 0).astype(
            jnp.float32)
        key = iota_f + 0.5 * sig
        neg_inf = jnp.float32(-jnp.inf)
        big = jnp.float32(n_exp)
        vs = []
        for _ in range(_TOP_K):
            m = jnp.max(work, axis=0, keepdims=True)
            v = jnp.min(jnp.where(work == m, key, big), axis=0, keepdims=True)
            vs.append(v)
            work = jnp.where(key == v, neg_inf, work)
        vmat = jnp.concatenate(vs, axis=0)
        idx_f = jnp.floor(vmat)
        wmat = 2.0 * (vmat - idx_f)
        wout_ref[...] = wmat / jnp.sum(wmat, axis=0, keepdims=True)
        iout_ref[...] = idx_f.astype(jnp.int32)


def kernel(x, W, expert_bias):
    t, d = x.shape
    e = W.shape[0]
    nblk = t // _BLOCK_T
    bias2d = expert_bias.reshape(e, 1)
    wout_t, iout_t = pl.pallas_call(
        _gate_kernel,
        grid=(nblk + 1,),
        in_specs=[
            pl.BlockSpec((_BLOCK_T, d), lambda j: (jnp.minimum(j, nblk - 1), 0)),
            pl.BlockSpec((e, d), lambda j: (0, 0)),
            pl.BlockSpec((e, 1), lambda j: (0, 0)),
        ],
        out_specs=[
            pl.BlockSpec((_TOP_K, _BLOCK_T), lambda j: (0, jnp.maximum(j - 1, 0))),
            pl.BlockSpec((_TOP_K, _BLOCK_T), lambda j: (0, jnp.maximum(j - 1, 0))),
        ],
        out_shape=[
            jax.ShapeDtypeStruct((_TOP_K, t), jnp.float32),
            jax.ShapeDtypeStruct((_TOP_K, t), jnp.int32),
        ],
        scratch_shapes=[pltpu.VMEM((2, e, _BLOCK_T), jnp.float32)],
    )(x, W, bias2d)
    return (wout_t.T, iout_t.T)
